# baseline (device time: 21203 ns/iter reference)
import jax
import jax.numpy as jnp
from jax import lax
from jax.experimental import pallas as pl
from jax.experimental.pallas import tpu as pltpu

N_DEV = 8
TAPS = 4
HALO = TAPS - 1


def kernel(x, k):
    b, s, c = x.shape

    def body(x_ref, k_ref, out_ref, halo_ref, send_sem, recv_sem):
        my = lax.axis_index("i")

        @pl.when(my < N_DEV - 1)
        def _():
            send = pltpu.make_async_remote_copy(
                src_ref=x_ref.at[:, pl.ds(s - HALO, HALO), :],
                dst_ref=halo_ref,
                send_sem=send_sem,
                recv_sem=recv_sem,
                device_id=(my + 1,),
                device_id_type=pl.DeviceIdType.MESH,
            )
            send.start()
            send.wait_send()

        @pl.when(my == 0)
        def _():
            halo_ref[...] = jnp.zeros((b, HALO, c), jnp.float32)

        @pl.when(my > 0)
        def _():
            recv = pltpu.make_async_remote_copy(
                src_ref=x_ref.at[:, pl.ds(s - HALO, HALO), :],
                dst_ref=halo_ref,
                send_sem=send_sem,
                recv_sem=recv_sem,
                device_id=(my - 1,),
                device_id_type=pl.DeviceIdType.MESH,
            )
            recv.wait_recv()

        xv = x_ref[...]
        hv = halo_ref[...]
        ext = jnp.concatenate([hv, xv], axis=1)
        acc = ext[:, 0:s, :] * k_ref[0][None, None, :]
        for t in range(1, TAPS):
            acc = acc + ext[:, t:t + s, :] * k_ref[t][None, None, :]
        out_ref[...] = acc * jax.nn.sigmoid(acc)

    return pl.pallas_call(
        body,
        out_shape=jax.ShapeDtypeStruct((b, s, c), jnp.float32),
        in_specs=[
            pl.BlockSpec(memory_space=pltpu.VMEM),
            pl.BlockSpec(memory_space=pltpu.VMEM),
        ],
        out_specs=pl.BlockSpec(memory_space=pltpu.VMEM),
        scratch_shapes=[
            pltpu.VMEM((b, HALO, c), jnp.float32),
            pltpu.SemaphoreType.DMA,
            pltpu.SemaphoreType.DMA,
        ],
    )(x, k)
